# baseline (speedup 1.0000x reference)
import jax
import jax.numpy as jnp
from jax import lax
from jax.experimental import pallas as pl
from jax.experimental.pallas import tpu as pltpu
from jax.experimental.pallas import tpu_sc as plsc

NC = 2
NS = 16
L = 16
NW = NC * NS

N_T = 128
T0, T1 = 300, 600
D = 256 * 9 * 16
SPLIT = 8
DS = D // SPLIT
TASKS = N_T // NS


def _sc_body(t_hbm, vg0_hbm, vg1_hbm, out_hbm,
             t_v, l0_v, r0_v, wl0_v, wr0_v, l1_v, r1_v, wl1_v, wr1_v,
             buf_v, sem):
    wid = lax.axis_index("s") * NC + lax.axis_index("c")
    lane = lax.broadcasted_iota(jnp.int32, (L,), 0)

    pltpu.sync_copy(t_hbm, t_v)
    for c in range(N_T // L):
        tv = t_v[pl.ds(c * L, L)]
        for tdim, l_v, r_v, wl_v, wr_v in ((T0, l0_v, r0_v, wl0_v, wr0_v),
                                           (T1, l1_v, r1_v, wl1_v, wr1_v)):
            inp = tv * float(tdim)
            li = (inp + 1e-6).astype(jnp.int32)
            li = jnp.minimum(li, tdim - 1)
            ri = jnp.minimum(li + 1, tdim - 1)
            lif = li.astype(jnp.float32)
            l_v[pl.ds(c * L, L)] = li
            r_v[pl.ds(c * L, L)] = ri
            wr_v[pl.ds(c * L, L)] = inp - lif
            wl_v[pl.ds(c * L, L)] = (lif + 1.0) - inp

    for g, vg_hbm, l_v, r_v, wl_v, wr_v in (
            (0, vg0_hbm, l0_v, r0_v, wl0_v, wr0_v),
            (1, vg1_hbm, l1_v, r1_v, wl1_v, wr1_v)):
        @pl.when((wid >= g * NS) & (wid < (g + 1) * NS))
        def _(vg_hbm=vg_hbm, l_v=l_v, r_v=r_v, wl_v=wl_v, wr_v=wr_v, g=g):
            base = (wid - g * NS) * TASKS
            for k in range(TASKS):
                ts = base + k
                tsv = jnp.broadcast_to(ts, (L,))
                li = plsc.load_gather(l_v, [tsv])
                ri = plsc.load_gather(r_v, [tsv])
                idx = jnp.where(lane < SPLIT,
                                li * SPLIT + lane,
                                ri * SPLIT + (lane - SPLIT))
                pltpu.async_copy(vg_hbm.at[idx], buf_v, sem).wait()
                wl = plsc.load_gather(wl_v, [tsv])
                wr = plsc.load_gather(wr_v, [tsv])
                for s in range(SPLIT):
                    def blend(j, carry, s=s):
                        off = j * L
                        vl = buf_v[s, pl.ds(off, L)]
                        vr = buf_v[s + SPLIT, pl.ds(off, L)]
                        buf_v[s, pl.ds(off, L)] = wl * vl + wr * vr
                        return carry

                    lax.fori_loop(0, DS // L, blend, 0)
                pltpu.sync_copy(buf_v.at[pl.ds(0, SPLIT)], out_hbm.at[ts, g])


def kernel(t, vg0, vg1):
    vg0f = vg0.reshape(T0 * SPLIT, DS)
    vg1f = vg1.reshape(T1 * SPLIT, DS)
    mesh = plsc.VectorSubcoreMesh(core_axis_name="c", subcore_axis_name="s")
    run = pl.kernel(
        _sc_body,
        out_type=jax.ShapeDtypeStruct((N_T, 2, SPLIT, DS), jnp.float32),
        mesh=mesh,
        compiler_params=pltpu.CompilerParams(needs_layout_passes=False),
        scratch_types=[
            pltpu.VMEM((N_T,), jnp.float32),
            pltpu.VMEM((N_T,), jnp.int32),
            pltpu.VMEM((N_T,), jnp.int32),
            pltpu.VMEM((N_T,), jnp.float32),
            pltpu.VMEM((N_T,), jnp.float32),
            pltpu.VMEM((N_T,), jnp.int32),
            pltpu.VMEM((N_T,), jnp.int32),
            pltpu.VMEM((N_T,), jnp.float32),
            pltpu.VMEM((N_T,), jnp.float32),
            pltpu.VMEM((2 * SPLIT, DS), jnp.float32),
            pltpu.SemaphoreType.DMA,
        ],
    )
    out = run(t, vg0f, vg1f)
    return out.reshape(N_T, 2 * 256, 9, 16)


# speedup vs baseline: 1.0670x; 1.0670x over previous
import jax
import jax.numpy as jnp
from jax import lax
from jax.experimental import pallas as pl
from jax.experimental.pallas import tpu as pltpu
from jax.experimental.pallas import tpu_sc as plsc

NC = 2
NS = 16
L = 16
NW = NC * NS

N_T = 128
T0, T1 = 300, 600
D = 256 * 9 * 16
VR = 16
DS = D // VR
HALF = VR // 2
TASKS = N_T // NS
CHUNKS = 2 * TASKS
NBUF = 3


def _sc_body(t_hbm, vg0_hbm, vg1_hbm, out_hbm,
             t_v, l0_v, r0_v, wl0_v, wr0_v, l1_v, r1_v, wl1_v, wr1_v,
             b0, b1, b2, g0, g1, g2, o0, o1, o2):
    wid = lax.axis_index("s") * NC + lax.axis_index("c")
    lane = lax.broadcasted_iota(jnp.int32, (L,), 0)
    bufs = (b0, b1, b2)
    gsems = (g0, g1, g2)
    osems = (o0, o1, o2)

    pltpu.sync_copy(t_hbm, t_v)
    for c in range(N_T // L):
        tv = t_v[pl.ds(c * L, L)]
        for tdim, l_v, r_v, wl_v, wr_v in ((T0, l0_v, r0_v, wl0_v, wr0_v),
                                           (T1, l1_v, r1_v, wl1_v, wr1_v)):
            inp = tv * float(tdim)
            li = (inp + 1e-6).astype(jnp.int32)
            li = jnp.minimum(li, tdim - 1)
            ri = jnp.minimum(li + 1, tdim - 1)
            lif = li.astype(jnp.float32)
            l_v[pl.ds(c * L, L)] = li
            r_v[pl.ds(c * L, L)] = ri
            wr_v[pl.ds(c * L, L)] = inp - lif
            wl_v[pl.ds(c * L, L)] = (lif + 1.0) - inp

    for g, vg_hbm, l_v, r_v, wl_v, wr_v in (
            (0, vg0_hbm, l0_v, r0_v, wl0_v, wr0_v),
            (1, vg1_hbm, l1_v, r1_v, wl1_v, wr1_v)):
        @pl.when((wid >= g * NS) & (wid < (g + 1) * NS))
        def _(vg_hbm=vg_hbm, l_v=l_v, r_v=r_v, wl_v=wl_v, wr_v=wr_v, g=g):
            base = (wid - g * NS) * TASKS

            def idx_for(k):
                task, h = divmod(k, 2)
                tsv = jnp.broadcast_to(base + task, (L,))
                li = plsc.load_gather(l_v, [tsv])
                ri = plsc.load_gather(r_v, [tsv])
                return jnp.where(lane < HALF,
                                 li * VR + h * HALF + lane,
                                 ri * VR + h * HALF + (lane - HALF))

            gh = [None] * NBUF
            oh = [None] * NBUF
            gh[0] = pltpu.async_copy(vg_hbm.at[idx_for(0)], bufs[0], gsems[0])
            for k in range(CHUNKS):
                s = k % NBUF
                if k + 1 < CHUNKS:
                    sn = (k + 1) % NBUF
                    if oh[sn] is not None:
                        oh[sn].wait()
                    gh[sn] = pltpu.async_copy(vg_hbm.at[idx_for(k + 1)],
                                              bufs[sn], gsems[sn])
                gh[s].wait()
                task, h = divmod(k, 2)
                tsv = jnp.broadcast_to(base + task, (L,))
                wl = plsc.load_gather(wl_v, [tsv])
                wr = plsc.load_gather(wr_v, [tsv])
                buf = bufs[s]

                def blend(j, carry, buf=buf, wl=wl, wr=wr):
                    off = j * L
                    for srow in range(HALF):
                        vl = buf[srow, pl.ds(off, L)]
                        vr = buf[srow + HALF, pl.ds(off, L)]
                        buf[srow, pl.ds(off, L)] = wl * vl + wr * vr
                    return carry

                lax.fori_loop(0, DS // L, blend, 0)
                oh[s] = pltpu.async_copy(buf.at[pl.ds(0, HALF)],
                                         out_hbm.at[base + task, g, h],
                                         osems[s])
            for s in range(NBUF):
                if oh[s] is not None:
                    oh[s].wait()


def kernel(t, vg0, vg1):
    vg0f = vg0.reshape(T0 * VR, DS)
    vg1f = vg1.reshape(T1 * VR, DS)
    mesh = plsc.VectorSubcoreMesh(core_axis_name="c", subcore_axis_name="s")
    run = pl.kernel(
        _sc_body,
        out_type=jax.ShapeDtypeStruct((N_T, 2, 2, HALF, DS), jnp.float32),
        mesh=mesh,
        compiler_params=pltpu.CompilerParams(needs_layout_passes=False),
        scratch_types=[
            pltpu.VMEM((N_T,), jnp.float32),
            pltpu.VMEM((N_T,), jnp.int32),
            pltpu.VMEM((N_T,), jnp.int32),
            pltpu.VMEM((N_T,), jnp.float32),
            pltpu.VMEM((N_T,), jnp.float32),
            pltpu.VMEM((N_T,), jnp.int32),
            pltpu.VMEM((N_T,), jnp.int32),
            pltpu.VMEM((N_T,), jnp.float32),
            pltpu.VMEM((N_T,), jnp.float32),
            pltpu.VMEM((2 * HALF, DS), jnp.float32),
            pltpu.VMEM((2 * HALF, DS), jnp.float32),
            pltpu.VMEM((2 * HALF, DS), jnp.float32),
            pltpu.SemaphoreType.DMA,
            pltpu.SemaphoreType.DMA,
            pltpu.SemaphoreType.DMA,
            pltpu.SemaphoreType.DMA,
            pltpu.SemaphoreType.DMA,
            pltpu.SemaphoreType.DMA,
        ],
    )
    out = run(t, vg0f, vg1f)
    return out.reshape(N_T, 2 * 256, 9, 16)


# speedup vs baseline: 1.0891x; 1.0207x over previous
import jax
import jax.numpy as jnp
from jax import lax
from jax.experimental import pallas as pl
from jax.experimental.pallas import tpu as pltpu
from jax.experimental.pallas import tpu_sc as plsc

NC = 2
NS = 16
L = 16
NW = NC * NS

N_T = 128
T0, T1 = 300, 600
D = 256 * 9 * 16
VR = 16
DS = D // VR
HALF = VR // 2
TASKS = N_T // NW
NBUF = 3
CHUNKS = 2 * TASKS * 2


def _sc_body(t_hbm, vg0_hbm, vg1_hbm, out_hbm,
             t_v, b0, b1, b2, g0, g1, g2, o0, o1, o2):
    wid = lax.axis_index("s") * NC + lax.axis_index("c")
    base = wid * TASKS
    lane = lax.broadcasted_iota(jnp.int32, (L,), 0)
    bufs = (b0, b1, b2)
    gsems = (g0, g1, g2)
    osems = (o0, o1, o2)

    pltpu.sync_copy(t_hbm, t_v)

    sched = []
    for g, vg_hbm, tdim in ((0, vg0_hbm, T0), (1, vg1_hbm, T1)):
        for k in range(TASKS):
            for h in range(2):
                sched.append((vg_hbm, tdim, k, h, g))

    def chunk_state(c):
        vg_hbm, tdim, k, h, g = sched[c]
        tsv = jnp.broadcast_to(base + k, (L,))
        tv = plsc.load_gather(t_v, [tsv])
        inp = tv * float(tdim)
        li = jnp.minimum((inp + 1e-6).astype(jnp.int32), tdim - 1)
        ri = jnp.minimum(li + 1, tdim - 1)
        lif = li.astype(jnp.float32)
        wr = inp - lif
        wl = (lif + 1.0) - inp
        idx = jnp.where(lane < HALF,
                        li * VR + h * HALF + lane,
                        ri * VR + h * HALF + (lane - HALF))
        return vg_hbm, idx, wl, wr

    gh = [None] * NBUF
    oh = [None] * NBUF
    vg_hbm, idx, wl0, wr0 = chunk_state(0)
    gh[0] = pltpu.async_copy(vg_hbm.at[idx], bufs[0], gsems[0])
    weights = [(wl0, wr0)]
    for c in range(CHUNKS):
        s = c % NBUF
        if c + 1 < CHUNKS:
            sn = (c + 1) % NBUF
            if oh[sn] is not None:
                oh[sn].wait()
            vg_hbm, idx, wl_n, wr_n = chunk_state(c + 1)
            gh[sn] = pltpu.async_copy(vg_hbm.at[idx], bufs[sn], gsems[sn])
            weights.append((wl_n, wr_n))
        gh[s].wait()
        wl, wr = weights[c]
        buf = bufs[s]

        @plsc.parallel_loop(0, DS, step=L, unroll=4)
        def _(off, buf=buf, wl=wl, wr=wr):
            for srow in range(HALF):
                vl = buf[srow, pl.ds(off, L)]
                vr = buf[srow + HALF, pl.ds(off, L)]
                buf[srow, pl.ds(off, L)] = wl * vl + wr * vr

        _, _, k, h, g = sched[c]
        oh[s] = pltpu.async_copy(buf.at[pl.ds(0, HALF)],
                                 out_hbm.at[base + k, g, h],
                                 osems[s])
    for s in range(NBUF):
        if oh[s] is not None:
            oh[s].wait()


def kernel(t, vg0, vg1):
    vg0f = vg0.reshape(T0 * VR, DS)
    vg1f = vg1.reshape(T1 * VR, DS)
    mesh = plsc.VectorSubcoreMesh(core_axis_name="c", subcore_axis_name="s")
    run = pl.kernel(
        _sc_body,
        out_type=jax.ShapeDtypeStruct((N_T, 2, 2, HALF, DS), jnp.float32),
        mesh=mesh,
        compiler_params=pltpu.CompilerParams(needs_layout_passes=False),
        scratch_types=[
            pltpu.VMEM((N_T,), jnp.float32),
            pltpu.VMEM((2 * HALF, DS), jnp.float32),
            pltpu.VMEM((2 * HALF, DS), jnp.float32),
            pltpu.VMEM((2 * HALF, DS), jnp.float32),
            pltpu.SemaphoreType.DMA,
            pltpu.SemaphoreType.DMA,
            pltpu.SemaphoreType.DMA,
            pltpu.SemaphoreType.DMA,
            pltpu.SemaphoreType.DMA,
            pltpu.SemaphoreType.DMA,
        ],
    )
    out = run(t, vg0f, vg1f)
    return out.reshape(N_T, 2 * 256, 9, 16)
